# baseline (speedup 1.0000x reference)
import functools

import jax
import jax.numpy as jnp
from jax import lax
from jax.experimental import pallas as pl
from jax.experimental.pallas import tpu as pltpu
from jax.experimental.pallas import tpu_sc as plsc

N = 10000
NPAD = 10240
D = 128
O = 64
NC = 2
NS = 16
NTILE = NC * NS
C = 128
RPT = NPAD // NS
RB = RPT // C
BLK = 1280
GRID = NPAD // BLK


def _dot(a, b):
    return jnp.dot(a, b, precision=lax.Precision.HIGHEST,
                   preferred_element_type=jnp.float32)


def _lrelu(x):
    return jnp.where(x >= 0, x, 0.01 * x)


def _mm3_body(fu, fi, wf, wc, wcb, of, oc, ocb):
    u = fu[...]
    of[...] = _dot(u, wf[...])
    oc[...] = _dot(u, wc[...])
    ocb[...] = _dot(fi[...], wcb[...])


def _mm3(fu, fi, wf, wc, wcb):
    row = pl.BlockSpec((BLK, D), lambda i: (i, 0))
    full = pl.BlockSpec((D, D), lambda i: (0, 0))
    return pl.pallas_call(
        _mm3_body,
        grid=(GRID,),
        in_specs=[row, row, full, full, full],
        out_specs=[row, row, row],
        out_shape=[jax.ShapeDtypeStruct((NPAD, D), jnp.float32)] * 3,
    )(fu, fi, wf, wc, wcb)


def _mesh():
    return plsc.VectorSubcoreMesh(core_axis_name="c", subcore_axis_name="s",
                                  num_cores=NC, num_subcores=NS)


@functools.lru_cache(maxsize=None)
def _make_agg(width):
    out_type = [jax.ShapeDtypeStruct((NC * NPAD, width), jnp.float32)] * 3
    scratch = [
        pltpu.VMEM_SHARED((NPAD, width), jnp.float32),
        [pltpu.VMEM((C,), jnp.int32) for _ in range(2)],
        [pltpu.VMEM((C,), jnp.int32) for _ in range(2)],
        [pltpu.VMEM((C, width), jnp.float32) for _ in range(2)],
        [pltpu.SemaphoreType.DMA for _ in range(2)],
        [pltpu.SemaphoreType.DMA for _ in range(2)],
    ]

    def body(tf, tc, tcb, sf, df, sc_, dc, scb, dcb, zrows_h,
             af_o, ac_o, acb_o,
             acc_s, sidx, didx, rows, semI, semG):
        triples = [(tf, sf, df, af_o), (tc, sc_, dc, ac_o),
                   (tcb, scb, dcb, acb_o)]

        ept = sf.shape[0] // NTILE
        nchunk = ept // C
        c = lax.axis_index("c")
        s = lax.axis_index("s")
        tile = c * NS + s
        ebase = tile * ept
        rb = s * RPT
        ob = c * NPAD + s * RPT

        def idx_start(src, dst, g, b):
            off = pl.multiple_of(ebase + g * C, C)
            pltpu.async_copy(src.at[pl.ds(off, C)], sidx[b], semI[b])
            pltpu.async_copy(dst.at[pl.ds(off, C)], didx[b], semI[b])

        def idx_wait(src, dst, g, b):
            off = pl.multiple_of(ebase + g * C, C)
            pltpu.make_async_copy(src.at[pl.ds(off, C)], sidx[b],
                                  semI[b]).wait()
            pltpu.make_async_copy(dst.at[pl.ds(off, C)], didx[b],
                                  semI[b]).wait()

        for (tab, src, dst, acc_o) in triples:
            pltpu.sync_copy(zrows_h, acc_s.at[pl.ds(rb, RPT)])
            plsc.subcore_barrier()

            idx_start(src, dst, 0, 0)
            idx_wait(src, dst, 0, 0)
            pltpu.async_copy(tab.at[sidx[0]], rows[0], semG[0])
            idx_start(src, dst, 1, 1)

            @pl.loop(0, nchunk, step=2)
            def _(k):
                for b in range(2):
                    g = k + b
                    bn = 1 - b

                    @pl.when(g + 1 < nchunk)
                    def _():
                        idx_wait(src, dst, g + 1, bn)
                        pltpu.async_copy(tab.at[sidx[bn]], rows[bn],
                                         semG[bn])
                    pltpu.make_async_copy(tab.at[sidx[b]], rows[b],
                                          semG[b]).wait()
                    pltpu.sync_copy(rows[b], acc_s.at[didx[b]], add=True)

                    @pl.when(g + 2 < nchunk)
                    def _():
                        idx_start(src, dst, g + 2, b)

            plsc.subcore_barrier()

            pltpu.sync_copy(acc_s.at[pl.ds(rb, RPT)],
                            acc_o.at[pl.ds(ob, RPT)])

    return pl.kernel(body, out_type=out_type, mesh=_mesh(),
                     scratch_types=scratch,
                     compiler_params=pltpu.CompilerParams(
                         use_tc_tiling_on_sc=(width == D)))


@functools.lru_cache(maxsize=None)
def _make_deg():
    out_type = [jax.ShapeDtypeStruct((NC, NPAD, 8), jnp.float32)] * 3
    scratch = [
        pltpu.VMEM_SHARED((NPAD, 8), jnp.float32),
        [pltpu.VMEM((C,), jnp.int32) for _ in range(2)],
        pltpu.VMEM((C, 8), jnp.float32),
        [pltpu.SemaphoreType.DMA for _ in range(2)],
    ]

    def body(df, dc, dcb, ones8_h, zdeg_h, gf_o, gc_o, gcb_o,
             deg_s, didx, ones8_v, semI):
        ept = df.shape[0] // NTILE
        nchunk = ept // C
        c = lax.axis_index("c")
        s = lax.axis_index("s")
        ebase = (c * NS + s) * ept

        pltpu.sync_copy(ones8_h, ones8_v)

        def idx_start(dst, g, b):
            off = pl.multiple_of(ebase + g * C, C)
            pltpu.async_copy(dst.at[pl.ds(off, C)], didx[b], semI[b])

        def idx_wait(dst, g, b):
            off = pl.multiple_of(ebase + g * C, C)
            pltpu.make_async_copy(dst.at[pl.ds(off, C)], didx[b],
                                  semI[b]).wait()

        for (dst, deg_o) in [(df, gf_o), (dc, gc_o), (dcb, gcb_o)]:
            @pl.when(s == 0)
            def _():
                pltpu.sync_copy(zdeg_h, deg_s)
            plsc.subcore_barrier()

            idx_start(dst, 0, 0)
            idx_start(dst, 1, 1)

            @pl.loop(0, nchunk, step=2)
            def _(k):
                for b in range(2):
                    g = k + b
                    idx_wait(dst, g, b)
                    pltpu.sync_copy(ones8_v, deg_s.at[didx[b]], add=True)

                    @pl.when(g + 2 < nchunk)
                    def _():
                        idx_start(dst, g + 2, b)

            plsc.subcore_barrier()

            @pl.when(s == 0)
            def _():
                pltpu.sync_copy(deg_s, deg_o.at[c])

    return pl.kernel(body, out_type=out_type, mesh=_mesh(),
                     scratch_types=scratch,
                     compiler_params=pltpu.CompilerParams(
                         use_tc_tiling_on_sc=False))


def _comb1_body(af, acb, ac, gf, gcb, gc, bf, bcb, bc, wf, wc, wcb,
                of, oc, ocb):
    def mean(a_ref, g_ref, b_ref):
        sm = a_ref[0] + a_ref[1]
        dg = g_ref[0][:, :1] + g_ref[1][:, :1]
        return jnp.where(dg > 0, sm / jnp.maximum(dg, 1.0) + b_ref[...], 0.0)

    h_user = _lrelu(mean(af, gf, bf) + mean(acb, gcb, bcb))
    h_item = _lrelu(mean(ac, gc, bc))
    of[...] = _dot(h_user, wf[...])
    oc[...] = _dot(h_user, wc[...])
    ocb[...] = _dot(h_item, wcb[...])


def _comb1(af, acb, ac, gf, gcb, gc, bf, bcb, bc, wf, wc, wcb):
    acc = pl.BlockSpec((NC, BLK, D), lambda i: (0, i, 0))
    deg = pl.BlockSpec((NC, BLK, 8), lambda i: (0, i, 0))
    bia = pl.BlockSpec((1, D), lambda i: (0, 0))
    wsp = pl.BlockSpec((D, O), lambda i: (0, 0))
    row = pl.BlockSpec((BLK, O), lambda i: (i, 0))
    return pl.pallas_call(
        _comb1_body,
        grid=(GRID,),
        in_specs=[acc, acc, acc, deg, deg, deg, bia, bia, bia, wsp, wsp, wsp],
        out_specs=[row, row, row],
        out_shape=[jax.ShapeDtypeStruct((NPAD, O), jnp.float32)] * 3,
    )(af, acb, ac, gf, gcb, gc, bf, bcb, bc, wf, wc, wcb)


def _comb2_body(af, acb, ac, gf, gcb, gc, bf, bcb, bc, ou, oi):
    def mean(a_ref, g_ref, b_ref):
        sm = a_ref[0] + a_ref[1]
        dg = g_ref[0][:, :1] + g_ref[1][:, :1]
        return jnp.where(dg > 0, sm / jnp.maximum(dg, 1.0) + b_ref[...], 0.0)

    ou[...] = mean(af, gf, bf) + mean(acb, gcb, bcb)
    oi[...] = mean(ac, gc, bc)


def _comb2(af, acb, ac, gf, gcb, gc, bf, bcb, bc):
    acc = pl.BlockSpec((NC, BLK, O), lambda i: (0, i, 0))
    deg = pl.BlockSpec((NC, BLK, 8), lambda i: (0, i, 0))
    bia = pl.BlockSpec((1, O), lambda i: (0, 0))
    row = pl.BlockSpec((BLK, O), lambda i: (i, 0))
    return pl.pallas_call(
        _comb2_body,
        grid=(GRID,),
        in_specs=[acc, acc, acc, deg, deg, deg, bia, bia, bia],
        out_specs=[row, row],
        out_shape=[jax.ShapeDtypeStruct((NPAD, O), jnp.float32)] * 2,
    )(af, acb, ac, gf, gcb, gc, bf, bcb, bc)


def _pad_edges(e):
    ne = e.shape[1]
    per = -(-ne // NTILE)
    ep0 = per * NTILE
    src = jnp.concatenate([e[0], jnp.zeros((ep0 - ne,), jnp.int32)])
    dst = jnp.concatenate([e[1], jnp.full((ep0 - ne,), N, jnp.int32)])
    cpt = -(-per // C)
    cpt += cpt % 2
    ept = cpt * C
    pad = ept - per
    spad = jnp.arange(pad, dtype=jnp.int32) % N
    dpad = N + jnp.arange(pad, dtype=jnp.int32) % (NPAD - N)
    src = jnp.concatenate([src.reshape(NTILE, per),
                           jnp.broadcast_to(spad, (NTILE, pad))],
                          axis=1).reshape(-1)
    dst = jnp.concatenate([dst.reshape(NTILE, per),
                           jnp.broadcast_to(dpad, (NTILE, pad))],
                          axis=1).reshape(-1)
    return src, dst


@jax.jit
def kernel(feat_user, feat_item, edges_follows, edges_clicks,
           edges_clicked_by, W1_f, b1_f, W1_c, b1_c, W1_cb, b1_cb,
           W2_f, b2_f, W2_c, b2_c, W2_cb, b2_cb):
    fu = jnp.pad(feat_user, ((0, NPAD - N), (0, 0)))
    fi = jnp.pad(feat_item, ((0, NPAD - N), (0, 0)))
    sf, df = _pad_edges(edges_follows)
    sc_, dc = _pad_edges(edges_clicks)
    scb, dcb = _pad_edges(edges_clicked_by)

    zrD = jnp.zeros((RPT, D), jnp.float32)
    zrO = jnp.zeros((RPT, O), jnp.float32)
    ones8 = jnp.ones((C, 8), jnp.float32)
    zdeg = jnp.zeros((NPAD, 8), jnp.float32)

    gf, gc, gcb = _make_deg()(df, dc, dcb, ones8, zdeg)

    t1f, t1c, t1cb = _mm3(fu, fi, W1_f, W1_c, W1_cb)
    af, ac, acb = _make_agg(D)(t1f, t1c, t1cb, sf, df, sc_, dc, scb, dcb,
                               zrD)
    af = af.reshape(NC, NPAD, D)
    ac = ac.reshape(NC, NPAD, D)
    acb = acb.reshape(NC, NPAD, D)

    t2f, t2c, t2cb = _comb1(af, acb, ac, gf, gcb, gc,
                            b1_f.reshape(1, D), b1_cb.reshape(1, D),
                            b1_c.reshape(1, D), W2_f, W2_c, W2_cb)

    a2f, a2c, a2cb = _make_agg(O)(t2f, t2c, t2cb, sf, df, sc_, dc,
                                  scb, dcb, zrO)
    a2f = a2f.reshape(NC, NPAD, O)
    a2c = a2c.reshape(NC, NPAD, O)
    a2cb = a2cb.reshape(NC, NPAD, O)

    u2, i2 = _comb2(a2f, a2cb, a2c, gf, gcb, gc,
                    b2_f.reshape(1, O), b2_cb.reshape(1, O),
                    b2_c.reshape(1, O))
    return u2[:N], i2[:N]


# speedup vs baseline: 1.0316x; 1.0316x over previous
import functools

import jax
import jax.numpy as jnp
from jax import lax
from jax.experimental import pallas as pl
from jax.experimental.pallas import tpu as pltpu
from jax.experimental.pallas import tpu_sc as plsc

N = 10000
NPAD = 10240
D = 128
O = 64
NC = 2
NS = 16
NTILE = NC * NS
C = 128
RPT = NPAD // NS
RB = RPT // C
BLK = 1280
GRID = NPAD // BLK


def _dot(a, b):
    return jnp.dot(a, b, precision=lax.Precision.HIGHEST,
                   preferred_element_type=jnp.float32)


def _lrelu(x):
    return jnp.where(x >= 0, x, 0.01 * x)


def _mm3_body(fu, fi, wf, wc, wcb, of, oc, ocb):
    u = fu[...]
    of[...] = _dot(u, wf[...])
    oc[...] = _dot(u, wc[...])
    ocb[...] = _dot(fi[...], wcb[...])


def _mm3(fu, fi, wf, wc, wcb):
    row = pl.BlockSpec((BLK, D), lambda i: (i, 0))
    full = pl.BlockSpec((D, D), lambda i: (0, 0))
    return pl.pallas_call(
        _mm3_body,
        grid=(GRID,),
        in_specs=[row, row, full, full, full],
        out_specs=[row, row, row],
        out_shape=[jax.ShapeDtypeStruct((NPAD, D), jnp.float32)] * 3,
    )(fu, fi, wf, wc, wcb)


def _mesh():
    return plsc.VectorSubcoreMesh(core_axis_name="c", subcore_axis_name="s",
                                  num_cores=NC, num_subcores=NS)


@functools.lru_cache(maxsize=None)
def _make_agg(width):
    out_type = [jax.ShapeDtypeStruct((NC * NPAD, width), jnp.float32)] * 3
    scratch = [
        pltpu.VMEM_SHARED((NPAD, width), jnp.float32),
        [pltpu.VMEM((C,), jnp.int32) for _ in range(2)],
        [pltpu.VMEM((C,), jnp.int32) for _ in range(2)],
        [pltpu.VMEM((C, width), jnp.float32) for _ in range(2)],
        [pltpu.SemaphoreType.DMA for _ in range(2)],
        [pltpu.SemaphoreType.DMA for _ in range(2)],
        [pltpu.SemaphoreType.DMA for _ in range(2)],
    ]

    def body(tf, tc, tcb, sf, df, sc_, dc, scb, dcb, zrows_h,
             af_o, ac_o, acb_o,
             acc_s, sidx, didx, rows, semI, semG, semS):
        triples = [(tf, sf, df, af_o), (tc, sc_, dc, ac_o),
                   (tcb, scb, dcb, acb_o)]

        ept = sf.shape[0] // NTILE
        nchunk = ept // C
        c = lax.axis_index("c")
        s = lax.axis_index("s")
        tile = c * NS + s
        ebase = tile * ept
        rb = s * RPT
        ob = c * NPAD + s * RPT

        def idx_start(src, dst, g, b):
            off = pl.multiple_of(ebase + g * C, C)
            pltpu.async_copy(src.at[pl.ds(off, C)], sidx[b], semI[b])
            pltpu.async_copy(dst.at[pl.ds(off, C)], didx[b], semI[b])

        def idx_wait(src, dst, g, b):
            off = pl.multiple_of(ebase + g * C, C)
            pltpu.make_async_copy(src.at[pl.ds(off, C)], sidx[b],
                                  semI[b]).wait()
            pltpu.make_async_copy(dst.at[pl.ds(off, C)], didx[b],
                                  semI[b]).wait()

        for (tab, src, dst, acc_o) in triples:
            pltpu.sync_copy(zrows_h, rows[0])
            for k in range(RB):
                pltpu.sync_copy(rows[0], acc_s.at[pl.ds(rb + k * C, C)])
            plsc.subcore_barrier()

            idx_start(src, dst, 0, 0)
            idx_wait(src, dst, 0, 0)
            pltpu.async_copy(tab.at[sidx[0]], rows[0], semG[0])
            idx_start(src, dst, 1, 1)

            @pl.loop(0, nchunk, step=2)
            def _(k):
                for b in range(2):
                    g = k + b
                    bn = 1 - b

                    @pl.when(g + 1 < nchunk)
                    def _():
                        idx_wait(src, dst, g + 1, bn)
                        pltpu.async_copy(tab.at[sidx[bn]], rows[bn],
                                         semG[bn])
                    pltpu.make_async_copy(tab.at[sidx[b]], rows[b],
                                          semG[b]).wait()
                    pltpu.sync_copy(rows[b], acc_s.at[didx[b]], add=True)

                    @pl.when(g + 2 < nchunk)
                    def _():
                        idx_start(src, dst, g + 2, b)

            plsc.subcore_barrier()

            for k in range(RB):
                b = k % 2
                if k >= 2:
                    pltpu.make_async_copy(
                        rows[b], acc_o.at[pl.ds(ob + (k - 2) * C, C)],
                        semS[b]).wait()
                pltpu.sync_copy(acc_s.at[pl.ds(rb + k * C, C)], rows[b])
                pltpu.async_copy(rows[b], acc_o.at[pl.ds(ob + k * C, C)],
                                 semS[b])
            for k in range(RB - 2, RB):
                b = k % 2
                pltpu.make_async_copy(rows[b],
                                      acc_o.at[pl.ds(ob + k * C, C)],
                                      semS[b]).wait()

    return pl.kernel(body, out_type=out_type, mesh=_mesh(),
                     scratch_types=scratch,
                     compiler_params=pltpu.CompilerParams(
                         use_tc_tiling_on_sc=(width == D)))


@functools.lru_cache(maxsize=None)
def _make_deg():
    out_type = [jax.ShapeDtypeStruct((NC, NPAD, 8), jnp.float32)] * 3
    scratch = [
        pltpu.VMEM_SHARED((NPAD, 8), jnp.float32),
        [pltpu.VMEM((C,), jnp.int32) for _ in range(2)],
        pltpu.VMEM((C, 8), jnp.float32),
        [pltpu.SemaphoreType.DMA for _ in range(2)],
    ]

    def body(df, dc, dcb, ones8_h, zdeg_h, gf_o, gc_o, gcb_o,
             deg_s, didx, ones8_v, semI):
        ept = df.shape[0] // NTILE
        nchunk = ept // C
        c = lax.axis_index("c")
        s = lax.axis_index("s")
        ebase = (c * NS + s) * ept

        pltpu.sync_copy(ones8_h, ones8_v)

        def idx_start(dst, g, b):
            off = pl.multiple_of(ebase + g * C, C)
            pltpu.async_copy(dst.at[pl.ds(off, C)], didx[b], semI[b])

        def idx_wait(dst, g, b):
            off = pl.multiple_of(ebase + g * C, C)
            pltpu.make_async_copy(dst.at[pl.ds(off, C)], didx[b],
                                  semI[b]).wait()

        for (dst, deg_o) in [(df, gf_o), (dc, gc_o), (dcb, gcb_o)]:
            @pl.when(s == 0)
            def _():
                pltpu.sync_copy(zdeg_h, deg_s)
            plsc.subcore_barrier()

            idx_start(dst, 0, 0)
            idx_start(dst, 1, 1)

            @pl.loop(0, nchunk, step=2)
            def _(k):
                for b in range(2):
                    g = k + b
                    idx_wait(dst, g, b)
                    pltpu.sync_copy(ones8_v, deg_s.at[didx[b]], add=True)

                    @pl.when(g + 2 < nchunk)
                    def _():
                        idx_start(dst, g + 2, b)

            plsc.subcore_barrier()

            @pl.when(s == 0)
            def _():
                pltpu.sync_copy(deg_s, deg_o.at[c])

    return pl.kernel(body, out_type=out_type, mesh=_mesh(),
                     scratch_types=scratch,
                     compiler_params=pltpu.CompilerParams(
                         use_tc_tiling_on_sc=False))


def _comb1_body(af, acb, ac, gf, gcb, gc, bf, bcb, bc, wf, wc, wcb,
                of, oc, ocb):
    def mean(a_ref, g_ref, b_ref):
        sm = a_ref[0] + a_ref[1]
        dg = g_ref[0][:, :1] + g_ref[1][:, :1]
        return jnp.where(dg > 0, sm / jnp.maximum(dg, 1.0) + b_ref[...], 0.0)

    h_user = _lrelu(mean(af, gf, bf) + mean(acb, gcb, bcb))
    h_item = _lrelu(mean(ac, gc, bc))
    of[...] = _dot(h_user, wf[...])
    oc[...] = _dot(h_user, wc[...])
    ocb[...] = _dot(h_item, wcb[...])


def _comb1(af, acb, ac, gf, gcb, gc, bf, bcb, bc, wf, wc, wcb):
    acc = pl.BlockSpec((NC, BLK, D), lambda i: (0, i, 0))
    deg = pl.BlockSpec((NC, BLK, 8), lambda i: (0, i, 0))
    bia = pl.BlockSpec((1, D), lambda i: (0, 0))
    wsp = pl.BlockSpec((D, O), lambda i: (0, 0))
    row = pl.BlockSpec((BLK, O), lambda i: (i, 0))
    return pl.pallas_call(
        _comb1_body,
        grid=(GRID,),
        in_specs=[acc, acc, acc, deg, deg, deg, bia, bia, bia, wsp, wsp, wsp],
        out_specs=[row, row, row],
        out_shape=[jax.ShapeDtypeStruct((NPAD, O), jnp.float32)] * 3,
    )(af, acb, ac, gf, gcb, gc, bf, bcb, bc, wf, wc, wcb)


def _comb2_body(af, acb, ac, gf, gcb, gc, bf, bcb, bc, ou, oi):
    def mean(a_ref, g_ref, b_ref):
        sm = a_ref[0] + a_ref[1]
        dg = g_ref[0][:, :1] + g_ref[1][:, :1]
        return jnp.where(dg > 0, sm / jnp.maximum(dg, 1.0) + b_ref[...], 0.0)

    ou[...] = mean(af, gf, bf) + mean(acb, gcb, bcb)
    oi[...] = mean(ac, gc, bc)


def _comb2(af, acb, ac, gf, gcb, gc, bf, bcb, bc):
    acc = pl.BlockSpec((NC, BLK, O), lambda i: (0, i, 0))
    deg = pl.BlockSpec((NC, BLK, 8), lambda i: (0, i, 0))
    bia = pl.BlockSpec((1, O), lambda i: (0, 0))
    row = pl.BlockSpec((BLK, O), lambda i: (i, 0))
    return pl.pallas_call(
        _comb2_body,
        grid=(GRID,),
        in_specs=[acc, acc, acc, deg, deg, deg, bia, bia, bia],
        out_specs=[row, row],
        out_shape=[jax.ShapeDtypeStruct((NPAD, O), jnp.float32)] * 2,
    )(af, acb, ac, gf, gcb, gc, bf, bcb, bc)


def _pad_edges(e):
    ne = e.shape[1]
    per = -(-ne // NTILE)
    ep0 = per * NTILE
    src = jnp.concatenate([e[0], jnp.zeros((ep0 - ne,), jnp.int32)])
    dst = jnp.concatenate([e[1], jnp.full((ep0 - ne,), N, jnp.int32)])
    cpt = -(-per // C)
    cpt += cpt % 2
    ept = cpt * C
    pad = ept - per
    spad = jnp.arange(pad, dtype=jnp.int32) % N
    dpad = N + jnp.arange(pad, dtype=jnp.int32) % (NPAD - N)
    src = jnp.concatenate([src.reshape(NTILE, per),
                           jnp.broadcast_to(spad, (NTILE, pad))],
                          axis=1).reshape(-1)
    dst = jnp.concatenate([dst.reshape(NTILE, per),
                           jnp.broadcast_to(dpad, (NTILE, pad))],
                          axis=1).reshape(-1)
    return src, dst


@jax.jit
def kernel(feat_user, feat_item, edges_follows, edges_clicks,
           edges_clicked_by, W1_f, b1_f, W1_c, b1_c, W1_cb, b1_cb,
           W2_f, b2_f, W2_c, b2_c, W2_cb, b2_cb):
    fu = jnp.pad(feat_user, ((0, NPAD - N), (0, 0)))
    fi = jnp.pad(feat_item, ((0, NPAD - N), (0, 0)))
    sf, df = _pad_edges(edges_follows)
    sc_, dc = _pad_edges(edges_clicks)
    scb, dcb = _pad_edges(edges_clicked_by)

    zrD = jnp.zeros((C, D), jnp.float32)
    zrO = jnp.zeros((C, O), jnp.float32)
    ones8 = jnp.ones((C, 8), jnp.float32)
    zdeg = jnp.zeros((NPAD, 8), jnp.float32)

    gf, gc, gcb = _make_deg()(df, dc, dcb, ones8, zdeg)

    t1f, t1c, t1cb = _mm3(fu, fi, W1_f, W1_c, W1_cb)
    af, ac, acb = _make_agg(D)(t1f, t1c, t1cb, sf, df, sc_, dc, scb, dcb,
                               zrD)
    af = af.reshape(NC, NPAD, D)
    ac = ac.reshape(NC, NPAD, D)
    acb = acb.reshape(NC, NPAD, D)

    t2f, t2c, t2cb = _comb1(af, acb, ac, gf, gcb, gc,
                            b1_f.reshape(1, D), b1_cb.reshape(1, D),
                            b1_c.reshape(1, D), W2_f, W2_c, W2_cb)

    a2f, a2c, a2cb = _make_agg(O)(t2f, t2c, t2cb, sf, df, sc_, dc,
                                  scb, dcb, zrO)
    a2f = a2f.reshape(NC, NPAD, O)
    a2c = a2c.reshape(NC, NPAD, O)
    a2cb = a2cb.reshape(NC, NPAD, O)

    u2, i2 = _comb2(a2f, a2cb, a2c, gf, gcb, gc,
                    b2_f.reshape(1, O), b2_cb.reshape(1, O),
                    b2_c.reshape(1, O))
    return u2[:N], i2[:N]


# speedup vs baseline: 1.1037x; 1.0699x over previous
import functools

import jax
import jax.numpy as jnp
from jax import lax
from jax.experimental import pallas as pl
from jax.experimental.pallas import tpu as pltpu
from jax.experimental.pallas import tpu_sc as plsc

N = 10000
NPAD = 10240
D = 128
O = 64
NC = 2
NS = 16
NTILE = NC * NS
C = 128
RPT = NPAD // NS
RB = RPT // C
BLK = 1280
GRID = NPAD // BLK


def _dot(a, b):
    return jnp.dot(a, b, precision=lax.Precision.HIGHEST,
                   preferred_element_type=jnp.float32)


def _lrelu(x):
    return jnp.where(x >= 0, x, 0.01 * x)


def _mm3_body(fu, fi, wf, wc, wcb, of, oc, ocb):
    u = fu[...]
    of[...] = _dot(u, wf[...])
    oc[...] = _dot(u, wc[...])
    ocb[...] = _dot(fi[...], wcb[...])


def _mm3(fu, fi, wf, wc, wcb):
    row = pl.BlockSpec((BLK, D), lambda i: (i, 0))
    full = pl.BlockSpec((D, D), lambda i: (0, 0))
    return pl.pallas_call(
        _mm3_body,
        grid=(GRID,),
        in_specs=[row, row, full, full, full],
        out_specs=[row, row, row],
        out_shape=[jax.ShapeDtypeStruct((NPAD, D), jnp.float32)] * 3,
    )(fu, fi, wf, wc, wcb)


def _mesh():
    return plsc.VectorSubcoreMesh(core_axis_name="c", subcore_axis_name="s",
                                  num_cores=NC, num_subcores=NS)


@functools.lru_cache(maxsize=None)
def _make_agg(width):
    out_type = [jax.ShapeDtypeStruct((NC * NPAD, width), jnp.float32)] * 3
    scratch = [
        pltpu.VMEM_SHARED((NPAD, width), jnp.float32),
        [pltpu.VMEM((C,), jnp.int32) for _ in range(2)],
        [pltpu.VMEM((C,), jnp.int32) for _ in range(2)],
        [pltpu.VMEM((C, width), jnp.float32) for _ in range(2)],
        [pltpu.SemaphoreType.DMA for _ in range(2)],
        [pltpu.SemaphoreType.DMA for _ in range(2)],
        [pltpu.SemaphoreType.DMA for _ in range(2)],
    ]

    def body(tf, tc, tcb, sf, df, sc_, dc, scb, dcb, zrows_h,
             af_o, ac_o, acb_o,
             acc_s, sidx, didx, rows, semI, semG, semS):
        triples = [(tf, sf, df, af_o), (tc, sc_, dc, ac_o),
                   (tcb, scb, dcb, acb_o)]

        ept = sf.shape[0] // NTILE
        nchunk = ept // C
        c = lax.axis_index("c")
        s = lax.axis_index("s")
        tile = c * NS + s
        ebase = tile * ept
        rb = s * RPT
        ob = c * NPAD + s * RPT

        def idx_start(src, dst, g, b):
            off = pl.multiple_of(ebase + g * C, C)
            pltpu.async_copy(src.at[pl.ds(off, C)], sidx[b], semI[b])
            pltpu.async_copy(dst.at[pl.ds(off, C)], didx[b], semI[b])

        def idx_wait(src, dst, g, b):
            off = pl.multiple_of(ebase + g * C, C)
            pltpu.make_async_copy(src.at[pl.ds(off, C)], sidx[b],
                                  semI[b]).wait()
            pltpu.make_async_copy(dst.at[pl.ds(off, C)], didx[b],
                                  semI[b]).wait()

        for (tab, src, dst, acc_o) in triples:
            pltpu.sync_copy(zrows_h, rows[0])
            for k in range(RB):
                pltpu.sync_copy(rows[0], acc_s.at[pl.ds(rb + k * C, C)])
            plsc.subcore_barrier()

            idx_start(src, dst, 0, 0)
            idx_wait(src, dst, 0, 0)
            pltpu.async_copy(tab.at[sidx[0]], rows[0], semG[0])
            idx_start(src, dst, 1, 1)

            @pl.loop(0, nchunk, step=2)
            def _(k):
                for b in range(2):
                    g = k + b
                    bn = 1 - b

                    @pl.when(g + 1 < nchunk)
                    def _():
                        idx_wait(src, dst, g + 1, bn)
                        pltpu.async_copy(tab.at[sidx[bn]], rows[bn],
                                         semG[bn])
                    pltpu.make_async_copy(tab.at[sidx[b]], rows[b],
                                          semG[b]).wait()
                    pltpu.sync_copy(rows[b], acc_s.at[didx[b]], add=True)

                    @pl.when(g + 2 < nchunk)
                    def _():
                        idx_start(src, dst, g + 2, b)

            plsc.subcore_barrier()

            for k in range(RB):
                b = k % 2
                if k >= 2:
                    pltpu.make_async_copy(
                        rows[b], acc_o.at[pl.ds(ob + (k - 2) * C, C)],
                        semS[b]).wait()
                pltpu.sync_copy(acc_s.at[pl.ds(rb + k * C, C)], rows[b])
                pltpu.async_copy(rows[b], acc_o.at[pl.ds(ob + k * C, C)],
                                 semS[b])
            for k in range(RB - 2, RB):
                b = k % 2
                pltpu.make_async_copy(rows[b],
                                      acc_o.at[pl.ds(ob + k * C, C)],
                                      semS[b]).wait()

    return pl.kernel(body, out_type=out_type, mesh=_mesh(),
                     scratch_types=scratch,
                     compiler_params=pltpu.CompilerParams(
                         use_tc_tiling_on_sc=(width == D)))


@functools.lru_cache(maxsize=None)
def _make_deg():
    out_type = [jax.ShapeDtypeStruct((NC, NPAD, 8), jnp.float32)] * 3
    scratch = [
        pltpu.VMEM_SHARED((NPAD, 8), jnp.float32),
        [pltpu.VMEM((C,), jnp.int32) for _ in range(2)],
        pltpu.VMEM((C, 8), jnp.float32),
        [pltpu.SemaphoreType.DMA for _ in range(2)],
    ]

    def body(df, dc, dcb, ones8_h, zdeg_h, gf_o, gc_o, gcb_o,
             deg_s, didx, ones8_v, semI):
        ept = df.shape[0] // NTILE
        nchunk = ept // C
        c = lax.axis_index("c")
        s = lax.axis_index("s")
        ebase = (c * NS + s) * ept

        pltpu.sync_copy(ones8_h, ones8_v)

        def idx_start(dst, g, b):
            off = pl.multiple_of(ebase + g * C, C)
            pltpu.async_copy(dst.at[pl.ds(off, C)], didx[b], semI[b])

        def idx_wait(dst, g, b):
            off = pl.multiple_of(ebase + g * C, C)
            pltpu.make_async_copy(dst.at[pl.ds(off, C)], didx[b],
                                  semI[b]).wait()

        for (dst, deg_o) in [(df, gf_o), (dc, gc_o), (dcb, gcb_o)]:
            @pl.when(s == 0)
            def _():
                pltpu.sync_copy(zdeg_h, deg_s)
            plsc.subcore_barrier()

            idx_start(dst, 0, 0)
            idx_start(dst, 1, 1)

            @pl.loop(0, nchunk, step=2)
            def _(k):
                for b in range(2):
                    g = k + b
                    idx_wait(dst, g, b)
                    pltpu.sync_copy(ones8_v, deg_s.at[didx[b]], add=True)

                    @pl.when(g + 2 < nchunk)
                    def _():
                        idx_start(dst, g + 2, b)

            plsc.subcore_barrier()

            @pl.when(s == 0)
            def _():
                pltpu.sync_copy(deg_s, deg_o.at[c])

    return pl.kernel(body, out_type=out_type, mesh=_mesh(),
                     scratch_types=scratch,
                     compiler_params=pltpu.CompilerParams(
                         use_tc_tiling_on_sc=False))


def _comb1_body(af, acb, ac, gf, gcb, gc, bf, bcb, bc, wf, wc, wcb,
                of, oc, ocb):
    def mean(a_ref, g_ref, b_ref):
        sm = a_ref[0] + a_ref[1]
        dg = g_ref[0][:, :1] + g_ref[1][:, :1]
        return jnp.where(dg > 0, sm / jnp.maximum(dg, 1.0) + b_ref[...], 0.0)

    h_user = _lrelu(mean(af, gf, bf) + mean(acb, gcb, bcb))
    h_item = _lrelu(mean(ac, gc, bc))
    of[...] = _dot(h_user, wf[...])
    oc[...] = _dot(h_user, wc[...])
    ocb[...] = _dot(h_item, wcb[...])


def _comb1(af, acb, ac, gf, gcb, gc, bf, bcb, bc, wf, wc, wcb):
    acc = pl.BlockSpec((NC, BLK, D), lambda i: (0, i, 0))
    deg = pl.BlockSpec((NC, BLK, 8), lambda i: (0, i, 0))
    bia = pl.BlockSpec((1, D), lambda i: (0, 0))
    wsp = pl.BlockSpec((D, O), lambda i: (0, 0))
    row = pl.BlockSpec((BLK, O), lambda i: (i, 0))
    return pl.pallas_call(
        _comb1_body,
        grid=(GRID,),
        in_specs=[acc, acc, acc, deg, deg, deg, bia, bia, bia, wsp, wsp, wsp],
        out_specs=[row, row, row],
        out_shape=[jax.ShapeDtypeStruct((NPAD, O), jnp.float32)] * 3,
    )(af, acb, ac, gf, gcb, gc, bf, bcb, bc, wf, wc, wcb)


def _comb2_body(af, acb, ac, gf, gcb, gc, bf, bcb, bc, ou, oi):
    def mean_half(a_ref, g_ref, b_ref, h):
        sm = a_ref[0] + a_ref[1]
        dg = (g_ref[0] + g_ref[1])[:, 8 * h:8 * h + 1]
        smh = sm[:, 64 * h:64 * h + 64]
        return jnp.where(dg > 0, smh / jnp.maximum(dg, 1.0) + b_ref[...], 0.0)

    u = [mean_half(af, gf, bf, h) + mean_half(acb, gcb, bcb, h)
         for h in range(2)]
    i = [mean_half(ac, gc, bc, h) for h in range(2)]
    ou[...] = jnp.concatenate(u, axis=1)
    oi[...] = jnp.concatenate(i, axis=1)


def _comb2(af, acb, ac, gf, gcb, gc, bf, bcb, bc):
    B2 = 1000
    acc = pl.BlockSpec((NC, B2, 2 * O), lambda i: (0, i, 0))
    deg = pl.BlockSpec((NC, B2, 16), lambda i: (0, i, 0))
    bia = pl.BlockSpec((1, O), lambda i: (0, 0))
    row = pl.BlockSpec((B2, 2 * O), lambda i: (i, 0))
    return pl.pallas_call(
        _comb2_body,
        grid=(N // (2 * B2),),
        in_specs=[acc, acc, acc, deg, deg, deg, bia, bia, bia],
        out_specs=[row, row],
        out_shape=[jax.ShapeDtypeStruct((N // 2, 2 * O), jnp.float32)] * 2,
    )(af, acb, ac, gf, gcb, gc, bf, bcb, bc)


def _pad_edges(e):
    ne = e.shape[1]
    per = -(-ne // NTILE)
    ep0 = per * NTILE
    src = jnp.concatenate([e[0], jnp.zeros((ep0 - ne,), jnp.int32)])
    dst = jnp.concatenate([e[1], jnp.full((ep0 - ne,), N, jnp.int32)])
    cpt = -(-per // C)
    cpt += cpt % 2
    ept = cpt * C
    pad = ept - per
    spad = jnp.arange(pad, dtype=jnp.int32) % N
    dpad = N + jnp.arange(pad, dtype=jnp.int32) % (NPAD - N)
    src = jnp.concatenate([src.reshape(NTILE, per),
                           jnp.broadcast_to(spad, (NTILE, pad))],
                          axis=1).reshape(-1)
    dst = jnp.concatenate([dst.reshape(NTILE, per),
                           jnp.broadcast_to(dpad, (NTILE, pad))],
                          axis=1).reshape(-1)
    return src, dst


@jax.jit
def kernel(feat_user, feat_item, edges_follows, edges_clicks,
           edges_clicked_by, W1_f, b1_f, W1_c, b1_c, W1_cb, b1_cb,
           W2_f, b2_f, W2_c, b2_c, W2_cb, b2_cb):
    fu = jnp.pad(feat_user, ((0, NPAD - N), (0, 0)))
    fi = jnp.pad(feat_item, ((0, NPAD - N), (0, 0)))
    sf, df = _pad_edges(edges_follows)
    sc_, dc = _pad_edges(edges_clicks)
    scb, dcb = _pad_edges(edges_clicked_by)

    zrD = jnp.zeros((C, D), jnp.float32)
    zrO = jnp.zeros((C, O), jnp.float32)
    ones8 = jnp.ones((C, 8), jnp.float32)
    zdeg = jnp.zeros((NPAD, 8), jnp.float32)

    gf, gc, gcb = _make_deg()(df, dc, dcb, ones8, zdeg)

    t1f, t1c, t1cb = _mm3(fu, fi, W1_f, W1_c, W1_cb)
    af, ac, acb = _make_agg(D)(t1f, t1c, t1cb, sf, df, sc_, dc, scb, dcb,
                               zrD)
    af = af.reshape(NC, NPAD, D)
    ac = ac.reshape(NC, NPAD, D)
    acb = acb.reshape(NC, NPAD, D)

    t2f, t2c, t2cb = _comb1(af, acb, ac, gf, gcb, gc,
                            b1_f.reshape(1, D), b1_cb.reshape(1, D),
                            b1_c.reshape(1, D), W2_f, W2_c, W2_cb)

    a2f, a2c, a2cb = _make_agg(O)(t2f, t2c, t2cb, sf, df, sc_, dc,
                                  scb, dcb, zrO)
    a2f = a2f.reshape(NC, NPAD // 2, 2 * O)
    a2c = a2c.reshape(NC, NPAD // 2, 2 * O)
    a2cb = a2cb.reshape(NC, NPAD // 2, 2 * O)
    gfp = gf.reshape(NC, NPAD // 2, 16)
    gcp = gc.reshape(NC, NPAD // 2, 16)
    gcbp = gcb.reshape(NC, NPAD // 2, 16)

    u2, i2 = _comb2(a2f, a2cb, a2c, gfp, gcbp, gcp,
                    b2_f.reshape(1, O), b2_cb.reshape(1, O),
                    b2_c.reshape(1, O))
    return u2.reshape(N, O), i2.reshape(N, O)
